# Initial kernel scaffold; baseline (speedup 1.0000x reference)
import jax
import jax.numpy as jnp
from jax.experimental import pallas as pl


def kernel(x, res0_w1, res0_b1, res0_aw1, res0_ab1, res0_w2, res0_b2, res0_aw2, res0_ab2, res1_w1, res1_b1, res1_aw1, res1_ab1, res1_w2, res1_b2, res1_aw2, res1_ab2, res2_w1, res2_b1, res2_aw1, res2_ab1, res2_w2, res2_b2, res2_aw2, res2_ab2, res3_w1, res3_b1, res3_aw1, res3_ab1, res3_w2, res3_b2, res3_aw2, res3_ab2, up0_w, up0_b, up0_gamma, up0_beta, up1_w, up1_b, up1_gamma, up1_beta, out_w, out_b):
    raise NotImplementedError("write your pallas kernel here")


# speedup vs baseline: 11.4454x; 11.4454x over previous
import functools

import jax
import jax.numpy as jnp
from jax.experimental import pallas as pl
from jax.experimental.pallas import tpu as pltpu

F32 = jnp.float32
_EPS = 1e-5


def _res_body(xp_ref, w_ref, b_ref, a_ref, m_ref, o_ref, xa, xb):
    mask = m_ref[...]
    xa[...] = xp_ref[0]
    xb[...] = jnp.zeros_like(xb)
    CM = 272
    for r in range(4):
        for c in range(2):
            src = xa if c == 0 else xb
            for mc in range(4):
                base = mc * CM
                acc = jnp.zeros((CM, 256), F32)
                for t in range(9):
                    o = (t // 3) * 34 + (t % 3) + base
                    acc = acc + jnp.dot(src[o:o + CM, :],
                                        w_ref[2 * r + c,
                                              t * 256:(t + 1) * 256, :],
                                        preferred_element_type=F32)
                o_ref[0, base:base + CM, :] = acc + b_ref[2 * r + c, :][None, :]
            y = o_ref[0]
            mean = jnp.sum(y * mask, axis=0, keepdims=True) * (1.0 / 1024.0)
            d = (y - mean) * mask
            var = jnp.sum(d * d, axis=0, keepdims=True) * (1.0 / 1024.0)
            aw = a_ref[0, 4 * r + 2 * c, :][None, :]
            ab = a_ref[0, 4 * r + 2 * c + 1, :][None, :]
            z = d * jax.lax.rsqrt(var + _EPS) * aw + ab
            if c == 0:
                xb[35:1123, :] = jnp.maximum(z, 0.0) * mask
            else:
                xa[35:1123, :] = z * mask + xa[35:1123, :]
    o_ref[0] = xa[35:1123, :]


def _up_body(xp_ref, w_ref, b_ref, g_ref, bt_ref, m_ref, o_ref, *,
             wp, cin, cout, m_out, cm, n_chunks, n_valid):
    for mc in range(n_chunks):
        base = mc * cm
        acc = jnp.zeros((cm, cout), F32)
        for t in range(25):
            o = (t // 5) * wp + (t % 5) + base
            acc = acc + jnp.dot(xp_ref[0, o:o + cm, :],
                                w_ref[t * cin:(t + 1) * cin, :],
                                preferred_element_type=F32)
        o_ref[0, base:base + cm, :] = acc + b_ref[...]
    mask = m_ref[...]
    y = o_ref[0]
    s = jnp.sum(jnp.sum(y * mask, axis=0, keepdims=True),
                axis=1, keepdims=True)
    mean = s * (1.0 / n_valid)
    cen = y - mean
    d = cen * mask
    ss = jnp.sum(jnp.sum(d * d, axis=0, keepdims=True),
                 axis=1, keepdims=True)
    std = jnp.sqrt(ss * (1.0 / (n_valid - 1)))
    z = cen / (std + _EPS) * g_ref[...] + bt_ref[...]
    o_ref[0] = jnp.maximum(z, 0.0)


def _final_body(xp_ref, w_ref, b_ref, o_ref):
    CM = 1072
    for mc in range(16):
        base = mc * CM
        acc = jnp.zeros((CM, 128), F32)
        for t in range(49):
            o = (t // 7) * 134 + (t % 7) + base
            acc = acc + jnp.dot(xp_ref[0, o:o + CM, :],
                                w_ref[t * 64:(t + 1) * 64, :],
                                preferred_element_type=F32)
        o_ref[0, base:base + CM, :] = jnp.tanh(acc + b_ref[...])


def _flat_pad(img, pad, rows):
    b, h, w, c = img.shape
    p = jnp.pad(img, ((0, 0), (pad, pad), (pad, pad), (0, 0)))
    p = p.reshape(b, (h + 2 * pad) * (w + 2 * pad), c)
    return jnp.pad(p, ((0, 0), (0, rows - p.shape[1]), (0, 0)))


def _col_mask(m_rows, wp, wv, c):
    col = jnp.arange(m_rows, dtype=jnp.int32) % wp
    return jnp.broadcast_to((col < wv).astype(F32)[:, None], (m_rows, c))


_CP = pltpu.CompilerParams(dimension_semantics=("parallel",),
                           vmem_limit_bytes=128 * 1024 * 1024)


def kernel(x, res0_w1, res0_b1, res0_aw1, res0_ab1, res0_w2, res0_b2, res0_aw2, res0_ab2, res1_w1, res1_b1, res1_aw1, res1_ab1, res1_w2, res1_b2, res1_aw2, res1_ab2, res2_w1, res2_b1, res2_aw1, res2_ab1, res2_w2, res2_b2, res2_aw2, res2_ab2, res3_w1, res3_b1, res3_aw1, res3_ab1, res3_w2, res3_b2, res3_aw2, res3_ab2, up0_w, up0_b, up0_gamma, up0_beta, up1_w, up1_b, up1_gamma, up1_beta, out_w, out_b):
    B = x.shape[0]

    xn = jnp.transpose(x, (0, 2, 3, 1))
    xp = _flat_pad(xn, 1, 1160)
    W = jnp.stack([res0_w1, res0_w2, res1_w1, res1_w2,
                   res2_w1, res2_w2, res3_w1, res3_w2])
    bres = jnp.stack([res0_b1, res0_b2, res1_b1, res1_b2,
                      res2_b1, res2_b2, res3_b1, res3_b2])
    A = jnp.stack([res0_aw1, res0_ab1, res0_aw2, res0_ab2,
                   res1_aw1, res1_ab1, res1_aw2, res1_ab2,
                   res2_aw1, res2_ab1, res2_aw2, res2_ab2,
                   res3_aw1, res3_ab1, res3_aw2, res3_ab2], axis=1)
    mres = _col_mask(1088, 34, 32, 256)
    y = pl.pallas_call(
        _res_body,
        out_shape=jax.ShapeDtypeStruct((B, 1088, 256), F32),
        grid=(B,),
        in_specs=[
            pl.BlockSpec((1, 1160, 256), lambda i: (i, 0, 0)),
            pl.BlockSpec((8, 2304, 256), lambda i: (0, 0, 0)),
            pl.BlockSpec((8, 256), lambda i: (0, 0)),
            pl.BlockSpec((1, 16, 256), lambda i: (i, 0, 0)),
            pl.BlockSpec((1088, 256), lambda i: (0, 0)),
        ],
        out_specs=pl.BlockSpec((1, 1088, 256), lambda i: (i, 0, 0)),
        scratch_shapes=[pltpu.VMEM((1160, 256), F32),
                        pltpu.VMEM((1160, 256), F32)],
        compiler_params=_CP,
    )(xp, W, bres, A, mres)
    img = y.reshape(B, 32, 34, 256)[:, :, :32, :]

    u = jnp.repeat(jnp.repeat(img, 2, axis=1), 2, axis=2)
    up_in = _flat_pad(u, 2, 4632)
    m0 = _col_mask(4352, 68, 64, 128)
    y0 = pl.pallas_call(
        functools.partial(_up_body, wp=68, cin=256, cout=128, m_out=4352,
                          cm=544, n_chunks=8, n_valid=64 * 64 * 128),
        out_shape=jax.ShapeDtypeStruct((B, 4352, 128), F32),
        grid=(B,),
        in_specs=[
            pl.BlockSpec((1, 4632, 256), lambda i: (i, 0, 0)),
            pl.BlockSpec((6400, 128), lambda i: (0, 0)),
            pl.BlockSpec((1, 128), lambda i: (0, 0)),
            pl.BlockSpec((1, 128), lambda i: (0, 0)),
            pl.BlockSpec((1, 128), lambda i: (0, 0)),
            pl.BlockSpec((4352, 128), lambda i: (0, 0)),
        ],
        out_specs=pl.BlockSpec((1, 4352, 128), lambda i: (i, 0, 0)),
        compiler_params=_CP,
    )(up_in, up0_w, up0_b[None, :], up0_gamma[None, :], up0_beta[None, :], m0)
    img0 = y0.reshape(B, 64, 68, 128)[:, :, :64, :]

    u1 = jnp.repeat(jnp.repeat(img0, 2, axis=1), 2, axis=2)
    up1_in = _flat_pad(u1, 2, 17432)
    m1 = _col_mask(16896, 132, 128, 64)
    y1 = pl.pallas_call(
        functools.partial(_up_body, wp=132, cin=128, cout=64, m_out=16896,
                          cm=1056, n_chunks=16, n_valid=128 * 128 * 64),
        out_shape=jax.ShapeDtypeStruct((B, 16896, 64), F32),
        grid=(B,),
        in_specs=[
            pl.BlockSpec((1, 17432, 128), lambda i: (i, 0, 0)),
            pl.BlockSpec((3200, 64), lambda i: (0, 0)),
            pl.BlockSpec((1, 64), lambda i: (0, 0)),
            pl.BlockSpec((1, 64), lambda i: (0, 0)),
            pl.BlockSpec((1, 64), lambda i: (0, 0)),
            pl.BlockSpec((16896, 64), lambda i: (0, 0)),
        ],
        out_specs=pl.BlockSpec((1, 16896, 64), lambda i: (i, 0, 0)),
        compiler_params=_CP,
    )(up1_in, up1_w, up1_b[None, :], up1_gamma[None, :], up1_beta[None, :], m1)
    img1 = y1.reshape(B, 128, 132, 64)[:, :, :128, :]

    f_in = _flat_pad(img1, 3, 17968)
    wf = jnp.pad(out_w, ((0, 0), (0, 125)))
    bf = jnp.pad(out_b, (0, 125))[None, :]
    yf = pl.pallas_call(
        _final_body,
        out_shape=jax.ShapeDtypeStruct((B, 17152, 128), F32),
        grid=(B,),
        in_specs=[
            pl.BlockSpec((1, 17968, 64), lambda i: (i, 0, 0)),
            pl.BlockSpec((3136, 128), lambda i: (0, 0)),
            pl.BlockSpec((1, 128), lambda i: (0, 0)),
        ],
        out_specs=pl.BlockSpec((1, 17152, 128), lambda i: (i, 0, 0)),
        compiler_params=_CP,
    )(f_in, wf, bf)
    out = yf.reshape(B, 128, 134, 128)[:, :, :128, :3]
    return jnp.transpose(out, (0, 3, 1, 2))


# speedup vs baseline: 12.2579x; 1.0710x over previous
import functools

import jax
import jax.numpy as jnp
from jax.experimental import pallas as pl
from jax.experimental.pallas import tpu as pltpu

F32 = jnp.float32
BF16 = jnp.bfloat16
_EPS = 1e-5


def _res_body(xp_ref, w_ref, b_ref, a_ref, m_ref, o_ref, xa, xb):
    mask = m_ref[...]
    xa[...] = xp_ref[0]
    xb[...] = jnp.zeros_like(xb)
    CM = 272
    for r in range(4):
        for c in range(2):
            src = xa if c == 0 else xb
            for mc in range(4):
                base = mc * CM
                acc = jnp.zeros((CM, 256), F32)
                for t in range(9):
                    o = (t // 3) * 34 + (t % 3) + base
                    acc = acc + jnp.dot(src[o:o + CM, :],
                                        w_ref[2 * r + c,
                                              t * 256:(t + 1) * 256, :],
                                        preferred_element_type=F32)
                o_ref[0, base:base + CM, :] = acc + b_ref[2 * r + c, :][None, :]
            y = o_ref[0]
            mean = jnp.sum(y * mask, axis=0, keepdims=True) * (1.0 / 1024.0)
            d = (y - mean) * mask
            var = jnp.sum(d * d, axis=0, keepdims=True) * (1.0 / 1024.0)
            aw = a_ref[0, 4 * r + 2 * c, :][None, :]
            ab = a_ref[0, 4 * r + 2 * c + 1, :][None, :]
            z = d * jax.lax.rsqrt(var + _EPS) * aw + ab
            if c == 0:
                xb[35:1123, :] = jnp.maximum(z, 0.0) * mask
            else:
                xa[35:1123, :] = z * mask + xa[35:1123, :]
    o_ref[0] = xa[35:1123, :]


def _up_body(xp_ref, w_ref, b_ref, g_ref, bt_ref, m_ref, o_ref, *,
             wp, cin, cout, m_out, cm, n_chunks, n_valid, groups, gstep):
    taps_per_row = groups // 5
    for mc in range(n_chunks):
        base = mc * cm
        acc = jnp.zeros((cm, cout), F32)
        for g in range(groups):
            ky, gx = g // taps_per_row, g % taps_per_row
            o = ky * wp + gstep * gx + base
            kdim = gstep * cin
            acc = acc + jnp.dot(xp_ref[0, o:o + cm, :],
                                w_ref[g * kdim:(g + 1) * kdim, :],
                                preferred_element_type=F32)
        o_ref[0, base:base + cm, :] = acc + b_ref[...]
    mask = m_ref[...]
    y = o_ref[0]
    s = jnp.sum(jnp.sum(y * mask, axis=0, keepdims=True),
                axis=1, keepdims=True)
    mean = s * (1.0 / n_valid)
    cen = y - mean
    d = cen * mask
    ss = jnp.sum(jnp.sum(d * d, axis=0, keepdims=True),
                 axis=1, keepdims=True)
    std = jnp.sqrt(ss * (1.0 / (n_valid - 1)))
    z = cen / (std + _EPS) * g_ref[...] + bt_ref[...]
    o_ref[0] = jnp.maximum(z, 0.0)


def _final_body(xp_ref, w_ref, b_ref, o_ref):
    CM = 1088
    for mc in range(16):
        base = mc * CM
        acc = jnp.zeros((CM, 128), F32)
        for g in range(14):
            ky, gx = g // 2, g % 2
            o = ky * 136 + 4 * gx + base
            acc = acc + jnp.dot(xp_ref[0, o:o + CM, :],
                                w_ref[g * 256:(g + 1) * 256, :],
                                preferred_element_type=F32)
        o_ref[0, base:base + CM, :] = jnp.tanh(acc + b_ref[...])


def _flat_pad(img, pad, rows):
    b, h, w, c = img.shape
    p = jnp.pad(img, ((0, 0), (pad, pad), (pad, pad), (0, 0)))
    p = p.reshape(b, (h + 2 * pad) * (w + 2 * pad), c)
    return jnp.pad(p, ((0, 0), (0, rows - p.shape[1]), (0, 0)))


def _col_mask(m_rows, wp, wv, c):
    col = jnp.arange(m_rows, dtype=jnp.int32) % wp
    return jnp.broadcast_to((col < wv).astype(F32)[:, None], (m_rows, c))


_CP = pltpu.CompilerParams(dimension_semantics=("parallel",),
                           vmem_limit_bytes=128 * 1024 * 1024)


def kernel(x, res0_w1, res0_b1, res0_aw1, res0_ab1, res0_w2, res0_b2, res0_aw2, res0_ab2, res1_w1, res1_b1, res1_aw1, res1_ab1, res1_w2, res1_b2, res1_aw2, res1_ab2, res2_w1, res2_b1, res2_aw1, res2_ab1, res2_w2, res2_b2, res2_aw2, res2_ab2, res3_w1, res3_b1, res3_aw1, res3_ab1, res3_w2, res3_b2, res3_aw2, res3_ab2, up0_w, up0_b, up0_gamma, up0_beta, up1_w, up1_b, up1_gamma, up1_beta, out_w, out_b):
    B = x.shape[0]

    xn = jnp.transpose(x, (0, 2, 3, 1))
    xp = _flat_pad(xn, 1, 1168)
    W = jnp.stack([res0_w1, res0_w2, res1_w1, res1_w2,
                   res2_w1, res2_w2, res3_w1, res3_w2])
    bres = jnp.stack([res0_b1, res0_b2, res1_b1, res1_b2,
                      res2_b1, res2_b2, res3_b1, res3_b2])
    A = jnp.stack([res0_aw1, res0_ab1, res0_aw2, res0_ab2,
                   res1_aw1, res1_ab1, res1_aw2, res1_ab2,
                   res2_aw1, res2_ab1, res2_aw2, res2_ab2,
                   res3_aw1, res3_ab1, res3_aw2, res3_ab2], axis=1)
    mres = _col_mask(1088, 34, 32, 256)
    y = pl.pallas_call(
        _res_body,
        out_shape=jax.ShapeDtypeStruct((B, 1088, 256), F32),
        grid=(B,),
        in_specs=[
            pl.BlockSpec((1, 1168, 256), lambda i: (i, 0, 0)),
            pl.BlockSpec((8, 2304, 256), lambda i: (0, 0, 0)),
            pl.BlockSpec((8, 256), lambda i: (0, 0)),
            pl.BlockSpec((1, 16, 256), lambda i: (i, 0, 0)),
            pl.BlockSpec((1088, 256), lambda i: (0, 0)),
        ],
        out_specs=pl.BlockSpec((1, 1088, 256), lambda i: (i, 0, 0)),
        scratch_shapes=[pltpu.VMEM((1168, 256), F32),
                        pltpu.VMEM((1168, 256), F32)],
        compiler_params=_CP,
    )(xp, W, bres, A, mres)
    img = y.reshape(B, 32, 34, 256)[:, :, :32, :]

    u = jnp.repeat(jnp.repeat(img, 2, axis=1), 2, axis=2)
    up_in = _flat_pad(u, 2, 4640)
    m0 = _col_mask(4352, 68, 64, 128)
    y0 = pl.pallas_call(
        functools.partial(_up_body, wp=68, cin=256, cout=128, m_out=4352,
                          cm=544, n_chunks=8, n_valid=64 * 64 * 128,
                          groups=25, gstep=1),
        out_shape=jax.ShapeDtypeStruct((B, 4352, 128), F32),
        grid=(B,),
        in_specs=[
            pl.BlockSpec((1, 4640, 256), lambda i: (i, 0, 0)),
            pl.BlockSpec((6400, 128), lambda i: (0, 0)),
            pl.BlockSpec((1, 128), lambda i: (0, 0)),
            pl.BlockSpec((1, 128), lambda i: (0, 0)),
            pl.BlockSpec((1, 128), lambda i: (0, 0)),
            pl.BlockSpec((4352, 128), lambda i: (0, 0)),
        ],
        out_specs=pl.BlockSpec((1, 4352, 128), lambda i: (i, 0, 0)),
        compiler_params=_CP,
    )(up_in, up0_w, up0_b[None, :], up0_gamma[None, :],
      up0_beta[None, :], m0)
    img0 = y0.reshape(B, 64, 68, 128)[:, :, :64, :]

    u1 = jnp.repeat(jnp.repeat(img0, 2, axis=1), 2, axis=2)
    up1_flat = _flat_pad(u1, 2, 17456).astype(BF16)
    x2 = jnp.concatenate([up1_flat[:, 0:17440, :], up1_flat[:, 1:17441, :]],
                         axis=2)
    w1r = up1_w.reshape(5, 5, 128, 64)
    w1r = jnp.pad(w1r, ((0, 0), (0, 1), (0, 0), (0, 0)))
    w2 = w1r.reshape(5, 3, 256, 64).reshape(3840, 64).astype(BF16)
    m1 = _col_mask(16896, 132, 128, 64)
    y1 = pl.pallas_call(
        functools.partial(_up_body, wp=132, cin=128, cout=64, m_out=16896,
                          cm=1056, n_chunks=16, n_valid=128 * 128 * 64,
                          groups=15, gstep=2),
        out_shape=jax.ShapeDtypeStruct((B, 16896, 64), F32),
        grid=(B,),
        in_specs=[
            pl.BlockSpec((1, 17440, 256), lambda i: (i, 0, 0)),
            pl.BlockSpec((3840, 64), lambda i: (0, 0)),
            pl.BlockSpec((1, 64), lambda i: (0, 0)),
            pl.BlockSpec((1, 64), lambda i: (0, 0)),
            pl.BlockSpec((1, 64), lambda i: (0, 0)),
            pl.BlockSpec((16896, 64), lambda i: (0, 0)),
        ],
        out_specs=pl.BlockSpec((1, 16896, 64), lambda i: (i, 0, 0)),
        compiler_params=_CP,
    )(x2, w2, up1_b[None, :], up1_gamma[None, :],
      up1_beta[None, :], m1)
    img1 = y1.reshape(B, 128, 132, 64)[:, :, :128, :]

    fp = jnp.pad(img1, ((0, 0), (3, 3), (3, 5), (0, 0)))
    fp = fp.reshape(B, 134 * 136, 64)
    fp = jnp.pad(fp, ((0, 0), (0, 18256 - 134 * 136), (0, 0))).astype(BF16)
    x4 = jnp.concatenate([fp[:, j:j + 18240, :] for j in range(4)], axis=2)
    wfr = out_w.reshape(7, 7, 64, 3)
    wfr = jnp.pad(wfr, ((0, 0), (0, 1), (0, 0), (0, 0)))
    wf = wfr.reshape(7, 2, 256, 3).reshape(3584, 3)
    wf = jnp.pad(wf, ((0, 0), (0, 125))).astype(BF16)
    bf = jnp.pad(out_b, (0, 125))[None, :]
    yf = pl.pallas_call(
        _final_body,
        out_shape=jax.ShapeDtypeStruct((B, 17408, 128), F32),
        grid=(B,),
        in_specs=[
            pl.BlockSpec((1, 18240, 256), lambda i: (i, 0, 0)),
            pl.BlockSpec((3584, 128), lambda i: (0, 0)),
            pl.BlockSpec((1, 128), lambda i: (0, 0)),
        ],
        out_specs=pl.BlockSpec((1, 17408, 128), lambda i: (i, 0, 0)),
        compiler_params=_CP,
    )(x4, wf, bf)
    out = yf.reshape(B, 128, 136, 128)[:, :, :128, :3]
    return jnp.transpose(out, (0, 3, 1, 2))


# speedup vs baseline: 13.3356x; 1.0879x over previous
import functools

import jax
import jax.numpy as jnp
from jax.experimental import pallas as pl
from jax.experimental.pallas import tpu as pltpu

F32 = jnp.float32
BF16 = jnp.bfloat16
_EPS = 1e-5


def _res_body(xp_ref, w_ref, b_ref, a_ref, m_ref, o_ref, xa, xb):
    mask = m_ref[...]
    xa[...] = xp_ref[0]
    xb[...] = jnp.zeros_like(xb)
    CM = 272
    for r in range(4):
        for c in range(2):
            src = xa if c == 0 else xb
            for mc in range(4):
                base = mc * CM
                acc = jnp.zeros((CM, 256), F32)
                for t in range(9):
                    o = (t // 3) * 34 + (t % 3) + base
                    acc = acc + jnp.dot(src[o:o + CM, :],
                                        w_ref[2 * r + c,
                                              t * 256:(t + 1) * 256, :],
                                        preferred_element_type=F32)
                o_ref[0, base:base + CM, :] = acc + b_ref[2 * r + c, :][None, :]
            y = o_ref[0]
            mean = jnp.sum(y * mask, axis=0, keepdims=True) * (1.0 / 1024.0)
            d = (y - mean) * mask
            var = jnp.sum(d * d, axis=0, keepdims=True) * (1.0 / 1024.0)
            aw = a_ref[0, 4 * r + 2 * c, :][None, :]
            ab = a_ref[0, 4 * r + 2 * c + 1, :][None, :]
            z = d * jax.lax.rsqrt(var + _EPS) * aw + ab
            if c == 0:
                xb[35:1123, :] = jnp.maximum(z, 0.0) * mask
            else:
                xa[35:1123, :] = z * mask + xa[35:1123, :]
    o_ref[0] = xa[35:1123, :]


def _up_body(xp_ref, w_ref, b_ref, g_ref, bt_ref, m_ref, o_ref, *,
             wp, kdim, ncout, cm, n_chunks, n_valid):
    for mc in range(n_chunks):
        base = mc * cm
        acc = jnp.zeros((cm, ncout), F32)
        for dy in range(3):
            o = dy * wp + base
            acc = acc + jnp.dot(xp_ref[0, o:o + cm, :],
                                w_ref[dy * kdim:(dy + 1) * kdim, :],
                                preferred_element_type=F32)
        o_ref[0, base:base + cm, :] = acc + b_ref[...]
    mask = m_ref[...]
    y = o_ref[0]
    s = jnp.sum(jnp.sum(y * mask, axis=0, keepdims=True),
                axis=1, keepdims=True)
    mean = s * (1.0 / n_valid)
    cen = y - mean
    d = cen * mask
    ss = jnp.sum(jnp.sum(d * d, axis=0, keepdims=True),
                 axis=1, keepdims=True)
    std = jnp.sqrt(ss * (1.0 / (n_valid - 1)))
    z = cen / (std + _EPS) * g_ref[...] + bt_ref[...]
    o_ref[0] = jnp.maximum(z, 0.0)


def _final_body(xp_ref, w_ref, b_ref, o_ref):
    CM = 1152
    for mc in range(16):
        base = mc * CM
        acc = jnp.zeros((CM, 128), F32)
        for ky in range(7):
            o = ky * 144 + base
            acc = acc + jnp.dot(xp_ref[0, o:o + CM, :],
                                w_ref[ky * 512:(ky + 1) * 512, :],
                                preferred_element_type=F32)
        o_ref[0, base:base + CM, :] = jnp.tanh(acc + b_ref[...]).astype(BF16)


def _flat_pad(img, pad, rows):
    b, h, w, c = img.shape
    p = jnp.pad(img, ((0, 0), (pad, pad), (pad, pad), (0, 0)))
    p = p.reshape(b, (h + 2 * pad) * (w + 2 * pad), c)
    return jnp.pad(p, ((0, 0), (0, rows - p.shape[1]), (0, 0)))


def _phase_pack(img, w_flat, wp, rows, rows_pad):
    b, h, w, c = img.shape
    cout = w_flat.shape[1]
    p = jnp.pad(img, ((0, 0), (1, 1), (1, wp - w - 1), (0, 0)))
    p = p.reshape(b, rows, c)
    p = jnp.pad(p, ((0, 0), (0, rows_pad - rows), (0, 0))).astype(BF16)
    x4 = jnp.concatenate([p[:, j:j + rows, :] for j in range(4)], axis=2)
    R = jnp.array([[[1, 1, 0, 0, 0], [0, 0, 1, 1, 0], [0, 0, 0, 0, 1]],
                   [[1, 0, 0, 0, 0], [0, 1, 1, 0, 0], [0, 0, 0, 1, 1]]], F32)
    w4 = w_flat.reshape(5, 5, c, cout)
    weff = jnp.einsum('pak,qbl,klcd->abpqcd', R, R, w4)
    weff = jnp.transpose(weff, (0, 1, 4, 2, 3, 5))
    weff = jnp.pad(weff, ((0, 0), (0, 1), (0, 0), (0, 0), (0, 0), (0, 0)))
    wpk = weff.reshape(3 * 4 * c, 4 * cout).astype(BF16)
    return x4, wpk


def _col_mask(m_rows, wp, wv, c):
    col = jnp.arange(m_rows, dtype=jnp.int32) % wp
    return jnp.broadcast_to((col < wv).astype(F32)[:, None], (m_rows, c))


_CP = pltpu.CompilerParams(dimension_semantics=("parallel",),
                           vmem_limit_bytes=128 * 1024 * 1024)


def kernel(x, res0_w1, res0_b1, res0_aw1, res0_ab1, res0_w2, res0_b2, res0_aw2, res0_ab2, res1_w1, res1_b1, res1_aw1, res1_ab1, res1_w2, res1_b2, res1_aw2, res1_ab2, res2_w1, res2_b1, res2_aw1, res2_ab1, res2_w2, res2_b2, res2_aw2, res2_ab2, res3_w1, res3_b1, res3_aw1, res3_ab1, res3_w2, res3_b2, res3_aw2, res3_ab2, up0_w, up0_b, up0_gamma, up0_beta, up1_w, up1_b, up1_gamma, up1_beta, out_w, out_b):
    B = x.shape[0]

    xn = jnp.transpose(x, (0, 2, 3, 1))
    xp = _flat_pad(xn, 1, 1168)
    W = jnp.stack([res0_w1, res0_w2, res1_w1, res1_w2,
                   res2_w1, res2_w2, res3_w1, res3_w2])
    bres = jnp.stack([res0_b1, res0_b2, res1_b1, res1_b2,
                      res2_b1, res2_b2, res3_b1, res3_b2])
    A = jnp.stack([res0_aw1, res0_ab1, res0_aw2, res0_ab2,
                   res1_aw1, res1_ab1, res1_aw2, res1_ab2,
                   res2_aw1, res2_ab1, res2_aw2, res2_ab2,
                   res3_aw1, res3_ab1, res3_aw2, res3_ab2], axis=1)
    mres = _col_mask(1088, 34, 32, 256)
    y = pl.pallas_call(
        _res_body,
        out_shape=jax.ShapeDtypeStruct((B, 1088, 256), F32),
        grid=(B,),
        in_specs=[
            pl.BlockSpec((1, 1168, 256), lambda i: (i, 0, 0)),
            pl.BlockSpec((8, 2304, 256), lambda i: (0, 0, 0)),
            pl.BlockSpec((8, 256), lambda i: (0, 0)),
            pl.BlockSpec((1, 16, 256), lambda i: (i, 0, 0)),
            pl.BlockSpec((1088, 256), lambda i: (0, 0)),
        ],
        out_specs=pl.BlockSpec((1, 1088, 256), lambda i: (i, 0, 0)),
        scratch_shapes=[pltpu.VMEM((1168, 256), F32),
                        pltpu.VMEM((1168, 256), F32)],
        compiler_params=_CP,
    )(xp, W, bres, A, mres)
    img = y.reshape(B, 32, 34, 256)[:, :, :32, :]

    x4_0, w0p = _phase_pack(img, up0_w, 48, 1632, 1648)
    m0 = _col_mask(1536, 48, 32, 512)
    y0 = pl.pallas_call(
        functools.partial(_up_body, wp=48, kdim=1024, ncout=512,
                          cm=192, n_chunks=8, n_valid=64 * 64 * 128),
        out_shape=jax.ShapeDtypeStruct((B, 1536, 512), F32),
        grid=(B,),
        in_specs=[
            pl.BlockSpec((1, 1632, 1024), lambda i: (i, 0, 0)),
            pl.BlockSpec((3072, 512), lambda i: (0, 0)),
            pl.BlockSpec((1, 512), lambda i: (0, 0)),
            pl.BlockSpec((1, 512), lambda i: (0, 0)),
            pl.BlockSpec((1, 512), lambda i: (0, 0)),
            pl.BlockSpec((1536, 512), lambda i: (0, 0)),
        ],
        out_specs=pl.BlockSpec((1, 1536, 512), lambda i: (i, 0, 0)),
        compiler_params=_CP,
    )(x4_0, w0p, jnp.tile(up0_b, 4)[None, :], jnp.tile(up0_gamma, 4)[None, :],
      jnp.tile(up0_beta, 4)[None, :], m0)
    img0 = y0.reshape(B, 32, 48, 2, 2, 128)[:, :, :32]
    img0 = jnp.transpose(img0, (0, 1, 3, 2, 4, 5)).reshape(B, 64, 64, 128)

    x4_1, w1p = _phase_pack(img0, up1_w, 80, 5280, 5296)
    m1 = _col_mask(5120, 80, 64, 256)
    y1 = pl.pallas_call(
        functools.partial(_up_body, wp=80, kdim=512, ncout=256,
                          cm=640, n_chunks=8, n_valid=128 * 128 * 64),
        out_shape=jax.ShapeDtypeStruct((B, 5120, 256), F32),
        grid=(B,),
        in_specs=[
            pl.BlockSpec((1, 5280, 512), lambda i: (i, 0, 0)),
            pl.BlockSpec((1536, 256), lambda i: (0, 0)),
            pl.BlockSpec((1, 256), lambda i: (0, 0)),
            pl.BlockSpec((1, 256), lambda i: (0, 0)),
            pl.BlockSpec((1, 256), lambda i: (0, 0)),
            pl.BlockSpec((5120, 256), lambda i: (0, 0)),
        ],
        out_specs=pl.BlockSpec((1, 5120, 256), lambda i: (i, 0, 0)),
        compiler_params=_CP,
    )(x4_1, w1p, jnp.tile(up1_b, 4)[None, :], jnp.tile(up1_gamma, 4)[None, :],
      jnp.tile(up1_beta, 4)[None, :], m1)
    img1 = y1.reshape(B, 64, 80, 2, 2, 64)[:, :, :64]
    img1 = jnp.transpose(img1, (0, 1, 3, 2, 4, 5)).reshape(B, 128, 128, 64)

    fp = jnp.pad(img1, ((0, 0), (3, 3), (3, 13), (0, 0)))
    fp = fp.reshape(B, 134 * 144, 64)
    fp = jnp.pad(fp, ((0, 0), (0, 19312 - 134 * 144), (0, 0))).astype(BF16)
    x8 = jnp.concatenate([fp[:, j:j + 19296, :] for j in range(8)], axis=2)
    wfr = out_w.reshape(7, 7, 64, 3)
    wfr = jnp.pad(wfr, ((0, 0), (0, 1), (0, 0), (0, 0)))
    wf = wfr.reshape(3584, 3)
    wf = jnp.pad(wf, ((0, 0), (0, 125))).astype(BF16)
    bf = jnp.pad(out_b, (0, 125))[None, :]
    yf = pl.pallas_call(
        _final_body,
        out_shape=jax.ShapeDtypeStruct((B, 18432, 128), BF16),
        grid=(B,),
        in_specs=[
            pl.BlockSpec((1, 19296, 512), lambda i: (i, 0, 0)),
            pl.BlockSpec((3584, 128), lambda i: (0, 0)),
            pl.BlockSpec((1, 128), lambda i: (0, 0)),
        ],
        out_specs=pl.BlockSpec((1, 18432, 128), lambda i: (i, 0, 0)),
        compiler_params=_CP,
    )(x8, wf, bf)
    out = yf.reshape(B, 128, 144, 128)[:, :, :128, :3].astype(F32)
    return jnp.transpose(out, (0, 3, 1, 2))


# speedup vs baseline: 17.9626x; 1.3470x over previous
import functools

import jax
import jax.numpy as jnp
from jax.experimental import pallas as pl
from jax.experimental.pallas import tpu as pltpu

F32 = jnp.float32
BF16 = jnp.bfloat16
_EPS = 1e-5


def _res_body(xp_ref, w_ref, b_ref, a_ref, m_ref, o_ref, xa, xb):
    mask = m_ref[...]
    xa[...] = xp_ref[0]
    xb[...] = jnp.zeros_like(xb)
    CM = 272
    for r in range(4):
        for c in range(2):
            src = xa if c == 0 else xb
            for mc in range(4):
                base = mc * CM
                acc = jnp.zeros((CM, 256), F32)
                for t in range(9):
                    o = (t // 3) * 34 + (t % 3) + base
                    acc = acc + jnp.dot(src[o:o + CM, :],
                                        w_ref[2 * r + c,
                                              t * 256:(t + 1) * 256, :],
                                        preferred_element_type=F32)
                o_ref[0, base:base + CM, :] = acc + b_ref[2 * r + c, :][None, :]
            y = o_ref[0]
            mean = jnp.sum(y * mask, axis=0, keepdims=True) * (1.0 / 1024.0)
            d = (y - mean) * mask
            var = jnp.sum(d * d, axis=0, keepdims=True) * (1.0 / 1024.0)
            aw = a_ref[0, 4 * r + 2 * c, :][None, :]
            ab = a_ref[0, 4 * r + 2 * c + 1, :][None, :]
            z = d * jax.lax.rsqrt(var + _EPS) * aw + ab
            if c == 0:
                xb[35:1123, :] = jnp.maximum(z, 0.0) * mask
            else:
                xa[35:1123, :] = z * mask + xa[35:1123, :]
    o_ref[0] = xa[35:1123, :]


def _up_body(xp_ref, w_ref, b_ref, g_ref, bt_ref, m_ref, o_ref, *,
             wp, kdim, ncout, cm, n_chunks, n_valid):
    for mc in range(n_chunks):
        base = mc * cm
        acc = jnp.zeros((cm, ncout), F32)
        for dy in range(3):
            o = dy * wp + base
            acc = acc + jnp.dot(xp_ref[0, o:o + cm, :],
                                w_ref[dy * kdim:(dy + 1) * kdim, :],
                                preferred_element_type=F32)
        o_ref[0, base:base + cm, :] = acc + b_ref[...]
    mask = m_ref[...]
    y = o_ref[0]
    s = jnp.sum(jnp.sum(y * mask, axis=0, keepdims=True),
                axis=1, keepdims=True)
    mean = s * (1.0 / n_valid)
    cen = y - mean
    d = cen * mask
    ss = jnp.sum(jnp.sum(d * d, axis=0, keepdims=True),
                 axis=1, keepdims=True)
    std = jnp.sqrt(ss * (1.0 / (n_valid - 1)))
    z = cen / (std + _EPS) * g_ref[...] + bt_ref[...]
    o_ref[0] = jnp.maximum(z, 0.0)


def _final_body(xp_ref, w_ref, b_ref, o_ref, x8s):
    for j2 in range(4):
        x8s[:, j2 * 128:(j2 + 1) * 128] = xp_ref[0, 2 * j2:2 * j2 + 19296, :]
    CM = 1152
    for mc in range(16):
        base = mc * CM
        acc = jnp.zeros((CM, 128), F32)
        for ky in range(7):
            o = ky * 144 + base
            acc = acc + jnp.dot(x8s[o:o + CM, :],
                                w_ref[ky * 512:(ky + 1) * 512, :],
                                preferred_element_type=F32)
        o_ref[0, base:base + CM, :] = jnp.tanh(acc + b_ref[...]).astype(BF16)


def _flat_pad(img, pad, rows):
    b, h, w, c = img.shape
    p = jnp.pad(img, ((0, 0), (pad, pad), (pad, pad), (0, 0)))
    p = p.reshape(b, (h + 2 * pad) * (w + 2 * pad), c)
    return jnp.pad(p, ((0, 0), (0, rows - p.shape[1]), (0, 0)))


def _phase_pack(img, w_flat, wp, rows, rows_pad):
    b, h, w, c = img.shape
    cout = w_flat.shape[1]
    p = jnp.pad(img, ((0, 0), (1, 1), (1, wp - w - 1), (0, 0)))
    p = p.reshape(b, rows, c)
    p = jnp.pad(p, ((0, 0), (0, rows_pad - rows), (0, 0))).astype(BF16)
    x4 = jnp.concatenate([p[:, j:j + rows, :] for j in range(4)], axis=2)
    R = jnp.array([[[1, 1, 0, 0, 0], [0, 0, 1, 1, 0], [0, 0, 0, 0, 1]],
                   [[1, 0, 0, 0, 0], [0, 1, 1, 0, 0], [0, 0, 0, 1, 1]]], F32)
    w4 = w_flat.reshape(5, 5, c, cout)
    weff = jnp.einsum('pak,qbl,klcd->abpqcd', R, R, w4)
    weff = jnp.transpose(weff, (0, 1, 4, 2, 3, 5))
    weff = jnp.pad(weff, ((0, 0), (0, 1), (0, 0), (0, 0), (0, 0), (0, 0)))
    wpk = weff.reshape(3 * 4 * c, 4 * cout).astype(BF16)
    return x4, wpk


def _col_mask(m_rows, wp, wv, c):
    col = jnp.arange(m_rows, dtype=jnp.int32) % wp
    return jnp.broadcast_to((col < wv).astype(F32)[:, None], (m_rows, c))


_CP = pltpu.CompilerParams(dimension_semantics=("parallel",),
                           vmem_limit_bytes=128 * 1024 * 1024)


def kernel(x, res0_w1, res0_b1, res0_aw1, res0_ab1, res0_w2, res0_b2, res0_aw2, res0_ab2, res1_w1, res1_b1, res1_aw1, res1_ab1, res1_w2, res1_b2, res1_aw2, res1_ab2, res2_w1, res2_b1, res2_aw1, res2_ab1, res2_w2, res2_b2, res2_aw2, res2_ab2, res3_w1, res3_b1, res3_aw1, res3_ab1, res3_w2, res3_b2, res3_aw2, res3_ab2, up0_w, up0_b, up0_gamma, up0_beta, up1_w, up1_b, up1_gamma, up1_beta, out_w, out_b):
    B = x.shape[0]

    xn = jnp.transpose(x, (0, 2, 3, 1))
    xp = _flat_pad(xn, 1, 1168)
    W = jnp.stack([res0_w1, res0_w2, res1_w1, res1_w2,
                   res2_w1, res2_w2, res3_w1, res3_w2])
    bres = jnp.stack([res0_b1, res0_b2, res1_b1, res1_b2,
                      res2_b1, res2_b2, res3_b1, res3_b2])
    A = jnp.stack([res0_aw1, res0_ab1, res0_aw2, res0_ab2,
                   res1_aw1, res1_ab1, res1_aw2, res1_ab2,
                   res2_aw1, res2_ab1, res2_aw2, res2_ab2,
                   res3_aw1, res3_ab1, res3_aw2, res3_ab2], axis=1)
    mres = _col_mask(1088, 34, 32, 256)
    y = pl.pallas_call(
        _res_body,
        out_shape=jax.ShapeDtypeStruct((B, 1088, 256), F32),
        grid=(B,),
        in_specs=[
            pl.BlockSpec((1, 1168, 256), lambda i: (i, 0, 0)),
            pl.BlockSpec((8, 2304, 256), lambda i: (0, 0, 0)),
            pl.BlockSpec((8, 256), lambda i: (0, 0)),
            pl.BlockSpec((1, 16, 256), lambda i: (i, 0, 0)),
            pl.BlockSpec((1088, 256), lambda i: (0, 0)),
        ],
        out_specs=pl.BlockSpec((1, 1088, 256), lambda i: (i, 0, 0)),
        scratch_shapes=[pltpu.VMEM((1168, 256), F32),
                        pltpu.VMEM((1168, 256), F32)],
        compiler_params=_CP,
    )(xp, W, bres, A, mres)
    img = y.reshape(B, 32, 34, 256)[:, :, :32, :]

    x4_0, w0p = _phase_pack(img, up0_w, 48, 1632, 1648)
    m0 = _col_mask(1536, 48, 32, 512)
    y0 = pl.pallas_call(
        functools.partial(_up_body, wp=48, kdim=1024, ncout=512,
                          cm=192, n_chunks=8, n_valid=64 * 64 * 128),
        out_shape=jax.ShapeDtypeStruct((B, 1536, 512), F32),
        grid=(B,),
        in_specs=[
            pl.BlockSpec((1, 1632, 1024), lambda i: (i, 0, 0)),
            pl.BlockSpec((3072, 512), lambda i: (0, 0)),
            pl.BlockSpec((1, 512), lambda i: (0, 0)),
            pl.BlockSpec((1, 512), lambda i: (0, 0)),
            pl.BlockSpec((1, 512), lambda i: (0, 0)),
            pl.BlockSpec((1536, 512), lambda i: (0, 0)),
        ],
        out_specs=pl.BlockSpec((1, 1536, 512), lambda i: (i, 0, 0)),
        compiler_params=_CP,
    )(x4_0, w0p, jnp.tile(up0_b, 4)[None, :], jnp.tile(up0_gamma, 4)[None, :],
      jnp.tile(up0_beta, 4)[None, :], m0)
    img0 = y0.reshape(B, 32, 48, 2, 2, 128)[:, :, :32]
    img0 = jnp.transpose(img0, (0, 1, 3, 2, 4, 5)).reshape(B, 64, 64, 128)

    x4_1, w1p = _phase_pack(img0, up1_w, 80, 5280, 5296)
    m1 = _col_mask(5120, 80, 64, 256)
    y1 = pl.pallas_call(
        functools.partial(_up_body, wp=80, kdim=512, ncout=256,
                          cm=640, n_chunks=8, n_valid=128 * 128 * 64),
        out_shape=jax.ShapeDtypeStruct((B, 5120, 256), F32),
        grid=(B,),
        in_specs=[
            pl.BlockSpec((1, 5280, 512), lambda i: (i, 0, 0)),
            pl.BlockSpec((1536, 256), lambda i: (0, 0)),
            pl.BlockSpec((1, 256), lambda i: (0, 0)),
            pl.BlockSpec((1, 256), lambda i: (0, 0)),
            pl.BlockSpec((1, 256), lambda i: (0, 0)),
            pl.BlockSpec((5120, 256), lambda i: (0, 0)),
        ],
        out_specs=pl.BlockSpec((1, 5120, 256), lambda i: (i, 0, 0)),
        compiler_params=_CP,
    )(x4_1, w1p, jnp.tile(up1_b, 4)[None, :], jnp.tile(up1_gamma, 4)[None, :],
      jnp.tile(up1_beta, 4)[None, :], m1)
    img1 = y1.reshape(B, 64, 80, 2, 2, 64)[:, :, :64]
    img1 = jnp.transpose(img1, (0, 1, 3, 2, 4, 5)).reshape(B, 128, 128, 64)

    fp = jnp.pad(img1, ((0, 0), (3, 3), (3, 13), (0, 0)))
    fp = fp.reshape(B, 134 * 144, 64)
    fp = jnp.pad(fp, ((0, 0), (0, 19328 - 134 * 144), (0, 0))).astype(BF16)
    x2 = jnp.concatenate([fp[:, 0:19312, :], fp[:, 1:19313, :]], axis=2)
    wfr = out_w.reshape(7, 7, 64, 3)
    wfr = jnp.pad(wfr, ((0, 0), (0, 1), (0, 0), (0, 0)))
    wf = wfr.reshape(3584, 3)
    wf = jnp.pad(wf, ((0, 0), (0, 125))).astype(BF16)
    bf = jnp.pad(out_b, (0, 125))[None, :]
    yf = pl.pallas_call(
        _final_body,
        out_shape=jax.ShapeDtypeStruct((B, 18432, 128), BF16),
        grid=(B,),
        in_specs=[
            pl.BlockSpec((1, 19312, 128), lambda i: (i, 0, 0)),
            pl.BlockSpec((3584, 128), lambda i: (0, 0)),
            pl.BlockSpec((1, 128), lambda i: (0, 0)),
        ],
        out_specs=pl.BlockSpec((1, 18432, 128), lambda i: (i, 0, 0)),
        scratch_shapes=[pltpu.VMEM((19296, 512), BF16)],
        compiler_params=_CP,
    )(x2, wf, bf)
    out = yf.reshape(B, 128, 144, 128)[:, :, :128, :3].astype(F32)
    return jnp.transpose(out, (0, 3, 1, 2))
